# baseline (speedup 1.0000x reference)
import functools

import jax
import jax.numpy as jnp
from jax import lax
from jax.experimental import pallas as pl
from jax.experimental.pallas import tpu as pltpu
from jax.experimental.pallas import tpu_sc as plsc

_N = 8192
_C = 32
_KS = (8192, 1024, 512)
_BETA = 0.25
_NC, _NS = 2, 16
_NW = _NC * _NS
_TPW = _N // _NW
_CHUNK = 128
_NCH = _TPW // _CHUNK
_T = 256
_G = _N // _T
_CP = 128


def _argmin_body(cb_ref, zt_ref, idx_ref, lsum_ref, csq_s, cb2_s):
    zt = zt_ref[0, 0]
    k = cb_ref.shape[0]
    t = zt.shape[1]

    @pl.when(pl.program_id(0) == 0)
    def _():
        cbm = cb_ref[...]
        csq_s[...] = jnp.sum(cbm * cbm, axis=1, keepdims=True)
        cb2_s[...] = (-2.0 * cbm).astype(jnp.bfloat16)

    csq = csq_s[...]
    zsq = jnp.sum(zt * zt, axis=0, keepdims=True)
    mm2 = lax.dot_general(cb2_s[...], zt.astype(jnp.bfloat16),
                          (((1,), (0,)), ((), ())),
                          preferred_element_type=jnp.float32)
    d = (zsq + csq) + mm2
    kc = min(k, 2048)
    acc_v = None
    for c in range(0, k, kc):
        blk = d[c:c + kc]
        m = jnp.min(blk, axis=0, keepdims=True)
        ii = c + lax.broadcasted_iota(jnp.int32, (kc, t), 0)
        ic = jnp.min(jnp.where(blk == m, ii, k), axis=0, keepdims=True)
        if acc_v is None:
            acc_v, acc_i, acc_d = m, ic, m
        else:
            upd = m < acc_v
            acc_i = jnp.where(upd, ic, acc_i)
            acc_d = jnp.where(upd, m, acc_d)
            acc_v = jnp.where(upd, m, acc_v)
        acc_v = acc_v.astype(jnp.bfloat16).astype(jnp.float32)
    idx_ref[0] = acc_i
    lsum_ref[0] = jnp.sum(acc_d, axis=1, keepdims=True)


def _argmin_layer(cb, xr, layer, t):
    k = cb.shape[0]
    g = _N // t
    per_b = 1024 // t
    idx, lsum = pl.pallas_call(
        _argmin_body,
        grid=(g,),
        in_specs=[pl.BlockSpec((k, _C), lambda i: (0, 0)),
                  pl.BlockSpec((1, 1, _C, t),
                               lambda i: (i // per_b, layer, 0, i % per_b))],
        out_specs=[pl.BlockSpec((1, 1, t), lambda i: (i, 0, 0)),
                   pl.BlockSpec((1, 1, 1), lambda i: (i, 0, 0))],
        out_shape=[jax.ShapeDtypeStruct((g, 1, t), jnp.int32),
                   jax.ShapeDtypeStruct((g, 1, 1), jnp.float32)],
        scratch_shapes=[pltpu.VMEM((k, 1), jnp.float32),
                        pltpu.VMEM((k, _C), jnp.bfloat16)],
    )(cb, xr)
    return idx.reshape(_N), lsum.reshape(g)


def _sc_body(cb_hbm, idxw, q_hbm, cnt, idx_v, rows_v, zer_v, one_v, sh, sem,
             *, k):
    cid = lax.axis_index("c")
    sid = lax.axis_index("s")
    wid = cid * _NS + sid

    def _zb(i, _):
        zer_v[pl.ds(i * 16, 16)] = jnp.zeros((16,), jnp.float32)
        return 0
    lax.fori_loop(0, k // 16, _zb, 0)

    def _ob(i, _):
        one_v[pl.ds(i * 16, 16)] = jnp.ones((16,), jnp.float32)
        return 0
    lax.fori_loop(0, _CHUNK // 16, _ob, 0)

    @pl.when(sid == 0)
    def _():
        pltpu.sync_copy(zer_v, sh)
    plsc.subcore_barrier()

    pltpu.sync_copy(idxw.at[wid], idx_v)
    for j in range(_NCH):
        pltpu.async_copy(cb_hbm.at[idx_v.at[j]], rows_v.at[j], sem).wait()
        pltpu.sync_copy(one_v, sh.at[idx_v.at[j]], add=True)
    pltpu.sync_copy(rows_v, q_hbm.at[wid])

    plsc.subcore_barrier()

    @pl.when(sid == 0)
    def _():
        pltpu.sync_copy(sh, cnt.at[cid])


@functools.cache
def _sc_gather_counts(k):
    return pl.kernel(
        functools.partial(_sc_body, k=k),
        out_type=[jax.ShapeDtypeStruct((_NW, _NCH, _CHUNK, _CP), jnp.float32),
                  jax.ShapeDtypeStruct((_NC, k), jnp.float32)],
        mesh=plsc.VectorSubcoreMesh(core_axis_name="c", subcore_axis_name="s",
                                    num_cores=_NC, num_subcores=_NS),
        scratch_types=[pltpu.VMEM((_NCH, _CHUNK), jnp.int32),
                       pltpu.VMEM((_NCH, _CHUNK, _CP), jnp.float32),
                       pltpu.VMEM((k,), jnp.float32),
                       pltpu.VMEM((_CHUNK,), jnp.float32),
                       pltpu.VMEM_SHARED((k,), jnp.float32),
                       pltpu.SemaphoreType.DMA],
    )


def _fin_body(c0_ref, c1_ref, c2_ref, ls_ref, out_ref):
    for i, ref in enumerate((c0_ref, c1_ref, c2_ref)):
        cnt = jnp.sum(ref[...], axis=0, keepdims=True)
        p = cnt * (1.0 / _N)
        ent = jnp.sum(p * jnp.log(p + 1e-10))
        out_ref[1, i] = jnp.exp(-ent)
        m = jnp.sum(ls_ref[i, :]) * (1.0 / (_N * _C))
        out_ref[0, i] = m + _BETA * m


def _finalize(c0, c1, c2, ls):
    return pl.pallas_call(
        _fin_body,
        in_specs=[pl.BlockSpec(c0.shape, lambda: (0, 0)),
                  pl.BlockSpec(c1.shape, lambda: (0, 0)),
                  pl.BlockSpec(c2.shape, lambda: (0, 0)),
                  pl.BlockSpec(ls.shape, lambda: (0, 0))],
        out_specs=pl.BlockSpec(memory_space=pltpu.MemorySpace.SMEM),
        out_shape=jax.ShapeDtypeStruct((2, 4), jnp.float32),
    )(c0, c1, c2, ls)


def kernel(x, cb0, cb1, cb2):
    cbs = (cb0, cb1, cb2)
    ts = (1024, 1024, 1024)
    res = {}
    xr = x.reshape(8, 3, _C, 1024)
    for i in (1, 2, 0):
        idx, ls = _argmin_layer(cbs[i], xr, i, ts[i])
        cbp = jnp.pad(cbs[i], ((0, 0), (0, _CP - _C)))
        q, c = _sc_gather_counts(_KS[i])(
            cbp, idx.reshape(_NW, _NCH, _CHUNK))
        res[i] = (idx, ls, q, c)
    idxs = [res[i][0] for i in range(3)]
    lsums = [jnp.pad(res[i][1], (0, 32 - res[i][1].shape[0]))
             for i in range(3)]
    qs_raw = [res[i][2] for i in range(3)]
    cnts = [res[i][3] for i in range(3)]

    ls4 = jnp.stack(lsums + [lsums[0]])
    fin = _finalize(*cnts, ls4)
    loss_cat = fin[0, :3]
    perplexity_cat = fin[1, :3]

    qs = [q.reshape(_N, _CP)[:, :_C].reshape(8, 32, 32, _C).transpose(0, 3, 1, 2)
          for q in qs_raw]
    quantized_cat = jnp.concatenate(qs, axis=1)
    indices_cat = jnp.stack([ix.reshape(8, 32, 32) for ix in idxs], axis=1)
    return quantized_cat, indices_cat, loss_cat, perplexity_cat


# speedup vs baseline: 1.0628x; 1.0628x over previous
import functools

import jax
import jax.numpy as jnp
from jax import lax
from jax.experimental import pallas as pl
from jax.experimental.pallas import tpu as pltpu
from jax.experimental.pallas import tpu_sc as plsc

_N = 8192
_C = 32
_KS = (8192, 1024, 512)
_BETA = 0.25
_NC, _NS = 2, 16
_NW = _NC * _NS
_TPW = _N // _NW
_CHUNK = 128
_NCH = _TPW // _CHUNK
_T = 256
_G = _N // _T
_CP = 32


def _argmin_body(cb_ref, zt_ref, idx_ref, lsum_ref, csq_s, cb2_s):
    zt = zt_ref[0]
    k = cb_ref.shape[0]
    t = zt.shape[1]

    @pl.when(pl.program_id(0) == 0)
    def _():
        cbm = cb_ref[...]
        csq_s[...] = jnp.sum(cbm * cbm, axis=1, keepdims=True)
        cb2_s[...] = (-2.0 * cbm).astype(jnp.bfloat16)

    csq = csq_s[...]
    zsq = jnp.sum(zt * zt, axis=0, keepdims=True)
    mm2 = lax.dot_general(cb2_s[...], zt.astype(jnp.bfloat16),
                          (((1,), (0,)), ((), ())),
                          preferred_element_type=jnp.float32)
    d = (zsq + csq) + mm2
    kc = min(k, 2048)
    acc_v = None
    for c in range(0, k, kc):
        blk = d[c:c + kc]
        m = jnp.min(blk, axis=0, keepdims=True)
        ii = c + lax.broadcasted_iota(jnp.int32, (kc, t), 0)
        ic = jnp.min(jnp.where(blk == m, ii, k), axis=0, keepdims=True)
        if acc_v is None:
            acc_v, acc_i, acc_d = m, ic, m
        else:
            upd = m < acc_v
            acc_i = jnp.where(upd, ic, acc_i)
            acc_d = jnp.where(upd, m, acc_d)
            acc_v = jnp.where(upd, m, acc_v)
        acc_v = acc_v.astype(jnp.bfloat16).astype(jnp.float32)
    idx_ref[0] = acc_i
    lsum_ref[0] = jnp.sum(acc_d, axis=1, keepdims=True)


def _argmin_layer(cb, xb, t):
    k = cb.shape[0]
    g = _N // t
    per_b = 1024 // t
    idx, lsum = pl.pallas_call(
        _argmin_body,
        grid=(g,),
        in_specs=[pl.BlockSpec((k, _C), lambda i: (0, 0)),
                  pl.BlockSpec((1, _C, t),
                               lambda i: (i // per_b, 0, i % per_b))],
        out_specs=[pl.BlockSpec((1, 1, t), lambda i: (i, 0, 0)),
                   pl.BlockSpec((1, 1, 1), lambda i: (i, 0, 0))],
        out_shape=[jax.ShapeDtypeStruct((g, 1, t), jnp.int32),
                   jax.ShapeDtypeStruct((g, 1, 1), jnp.float32)],
        scratch_shapes=[pltpu.VMEM((k, 1), jnp.float32),
                        pltpu.VMEM((k, _C), jnp.bfloat16)],
    )(cb, xb)
    return idx.reshape(_N), lsum.reshape(g)


def _sc_body(cb_hbm, idxw, q_hbm, cnt, idx_v, rows_v, zer_v, one_v, sh, sem,
             *, k):
    cid = lax.axis_index("c")
    sid = lax.axis_index("s")
    wid = cid * _NS + sid

    def _zb(i, _):
        zer_v[pl.ds(i * 16, 16)] = jnp.zeros((16,), jnp.float32)
        return 0
    lax.fori_loop(0, k // 16, _zb, 0)

    def _ob(i, _):
        one_v[pl.ds(i * 16, 16)] = jnp.ones((16,), jnp.float32)
        return 0
    lax.fori_loop(0, _CHUNK // 16, _ob, 0)

    @pl.when(sid == 0)
    def _():
        pltpu.sync_copy(zer_v, sh)
    plsc.subcore_barrier()

    pltpu.sync_copy(idxw.at[wid], idx_v)
    for j in range(_NCH):
        pltpu.async_copy(cb_hbm.at[idx_v.at[j]], rows_v.at[j], sem).wait()
        pltpu.sync_copy(one_v, sh.at[idx_v.at[j]], add=True)
    pltpu.sync_copy(rows_v, q_hbm.at[wid])

    plsc.subcore_barrier()

    @pl.when(sid == 0)
    def _():
        pltpu.sync_copy(sh, cnt.at[cid])


@functools.cache
def _sc_gather_counts(k):
    return pl.kernel(
        functools.partial(_sc_body, k=k),
        out_type=[jax.ShapeDtypeStruct((_NW, _NCH, _CHUNK, _CP), jnp.float32),
                  jax.ShapeDtypeStruct((_NC, k), jnp.float32)],
        mesh=plsc.VectorSubcoreMesh(core_axis_name="c", subcore_axis_name="s",
                                    num_cores=_NC, num_subcores=_NS),
        compiler_params=pltpu.CompilerParams(use_tc_tiling_on_sc=False),
        scratch_types=[pltpu.VMEM((_NCH, _CHUNK), jnp.int32),
                       pltpu.VMEM((_NCH, _CHUNK, _CP), jnp.float32),
                       pltpu.VMEM((k,), jnp.float32),
                       pltpu.VMEM((_CHUNK,), jnp.float32),
                       pltpu.VMEM_SHARED((k,), jnp.float32),
                       pltpu.SemaphoreType.DMA],
    )


def _fin_body(c0_ref, c1_ref, c2_ref, ls_ref, out_ref):
    for i, ref in enumerate((c0_ref, c1_ref, c2_ref)):
        cnt = jnp.sum(ref[...], axis=0, keepdims=True)
        p = cnt * (1.0 / _N)
        ent = jnp.sum(p * jnp.log(p + 1e-10))
        out_ref[1, i] = jnp.exp(-ent)
        m = jnp.sum(ls_ref[i, :]) * (1.0 / (_N * _C))
        out_ref[0, i] = m + _BETA * m


def _finalize(c0, c1, c2, ls):
    return pl.pallas_call(
        _fin_body,
        in_specs=[pl.BlockSpec(c0.shape, lambda: (0, 0)),
                  pl.BlockSpec(c1.shape, lambda: (0, 0)),
                  pl.BlockSpec(c2.shape, lambda: (0, 0)),
                  pl.BlockSpec(ls.shape, lambda: (0, 0))],
        out_specs=pl.BlockSpec(memory_space=pltpu.MemorySpace.SMEM),
        out_shape=jax.ShapeDtypeStruct((2, 4), jnp.float32),
    )(c0, c1, c2, ls)


def kernel(x, cb0, cb1, cb2):
    cbs = (cb0, cb1, cb2)
    ts = (1024, 1024, 1024)
    res = {}
    for i in (1, 2, 0):
        xb = x[:, _C * i:_C * (i + 1)].reshape(8, _C, 1024)
        idx, ls = _argmin_layer(cbs[i], xb, ts[i])
        q, c = _sc_gather_counts(_KS[i])(
            cbs[i], idx.reshape(_NW, _NCH, _CHUNK))
        res[i] = (idx, ls, q, c)
    idxs = [res[i][0] for i in range(3)]
    lsums = [jnp.pad(res[i][1], (0, 32 - res[i][1].shape[0]))
             for i in range(3)]
    qs_raw = [res[i][2] for i in range(3)]
    cnts = [res[i][3] for i in range(3)]

    ls4 = jnp.stack(lsums + [lsums[0]])
    fin = _finalize(*cnts, ls4)
    loss_cat = fin[0, :3]
    perplexity_cat = fin[1, :3]

    qs = [q.reshape(8, 32, 32, _C).transpose(0, 3, 1, 2) for q in qs_raw]
    quantized_cat = jnp.concatenate(qs, axis=1)
    indices_cat = jnp.stack([ix.reshape(8, 32, 32) for ix in idxs], axis=1)
    return quantized_cat, indices_cat, loss_cat, perplexity_cat
